# baseline (speedup 1.0000x reference)
import functools

import jax
import jax.numpy as jnp
from jax.experimental import pallas as pl
from jax.experimental.pallas import tpu as pltpu


def _softplus(x):
    return jnp.maximum(x, 0.0) + jnp.log1p(jnp.exp(-jnp.abs(x)))


def _flow_body(fwd_hbm, bwd_ref, w1_ref, b1_ref, w2_ref, b2_ref, out_ref,
               xf_vmem, dma_sem, *, nact, bb, t):
    i = pl.program_id(0)
    fwd_copy = pltpu.make_async_copy(
        fwd_hbm.at[pl.ds(i * bb, bb), :, 0, :], xf_vmem, dma_sem
    )
    fwd_copy.start()

    w1 = w1_ref[...]
    b1 = b1_ref[...]
    w2 = w2_ref[...]
    b2 = b2_ref[...]
    m = bb * t

    fin = None
    for a in range(nact):
        xb = bwd_ref[:, :, a + 1, :].reshape(m, -1)
        hb = jnp.maximum(jnp.dot(xb, w1, preferred_element_type=jnp.float32) + b1, 0.0)
        zb = jnp.dot(hb, w2, preferred_element_type=jnp.float32) + b2
        term = _softplus(zb[:, a : a + 1])
        fin = term if fin is None else fin + term

    fwd_copy.wait()
    xf = xf_vmem[...].reshape(m, -1)
    hf = jnp.maximum(jnp.dot(xf, w1, preferred_element_type=jnp.float32) + b1, 0.0)
    zf = jnp.dot(hf, w2, preferred_element_type=jnp.float32) + b2
    fout = jnp.sum(_softplus(zf), axis=1, keepdims=True)

    out_ref[:, :, 0:1] = fin.reshape(bb, t, 1)
    out_ref[:, :, 1:2] = fout.reshape(bb, t, 1)


@functools.partial(jax.jit, static_argnames=("interpret",))
def _flow_pallas(forward_edges, backward_edges, W1, b1, W2, b2, interpret=False):
    b, t, a1, d = forward_edges.shape
    nact = a1 - 1
    h = W1.shape[1]
    bb = 4

    out = pl.pallas_call(
        functools.partial(_flow_body, nact=nact, bb=bb, t=t),
        grid=(b // bb,),
        in_specs=[
            pl.BlockSpec(memory_space=pltpu.MemorySpace.HBM),
            pl.BlockSpec((bb, t, a1, d), lambda i: (i, 0, 0, 0)),
            pl.BlockSpec((d, h), lambda i: (0, 0)),
            pl.BlockSpec((1, h), lambda i: (0, 0)),
            pl.BlockSpec((h, nact), lambda i: (0, 0)),
            pl.BlockSpec((1, nact), lambda i: (0, 0)),
        ],
        out_specs=pl.BlockSpec((bb, t, 2), lambda i: (i, 0, 0)),
        out_shape=jax.ShapeDtypeStruct((b, t, 2), jnp.float32),
        scratch_shapes=[
            pltpu.VMEM((bb, t, d), jnp.float32),
            pltpu.SemaphoreType.DMA,
        ],
        compiler_params=pltpu.CompilerParams(
            dimension_semantics=("parallel",),
        ),
        interpret=interpret,
    )(forward_edges, backward_edges, W1, b1.reshape(1, h), W2, b2.reshape(1, nact))
    return out


def kernel(forward_edges, backward_edges, paths_reward, W1, b1, W2, b2, initial_flow):
    b, t, a1, d = forward_edges.shape
    fin_fout = _flow_pallas(forward_edges, backward_edges, W1, b1, W2, b2)
    r = paths_reward.reshape(b, t, 1)
    finit = jnp.broadcast_to(initial_flow.reshape(1, 1, 1), (b, t, 1)).astype(jnp.float32)
    return jnp.concatenate([fin_fout, r, finit], axis=-1)


# speedup vs baseline: 1.4891x; 1.4891x over previous
import functools

import jax
import jax.numpy as jnp
from jax.experimental import pallas as pl
from jax.experimental.pallas import tpu as pltpu


def _softplus(x):
    return jnp.maximum(x, 0.0) + jnp.log1p(jnp.exp(-jnp.abs(x)))


def _flow_body(fwd_ref, bwd_ref, w1_ref, b1_ref, w2_ref, b2_ref, out_ref,
               *, nact, bb, t):
    s = pl.program_id(1)
    m = bb * t
    w1 = w1_ref[...]
    b1 = b1_ref[...]
    w2 = w2_ref[...]
    b2 = b2_ref[...]

    @pl.when(s == 0)
    def _forward():
        x = fwd_ref[...].reshape(m, -1)
        h = jnp.maximum(jnp.dot(x, w1, preferred_element_type=jnp.float32) + b1, 0.0)
        z = jnp.dot(h, w2, preferred_element_type=jnp.float32) + b2
        fout = jnp.sum(_softplus(z), axis=1, keepdims=True)
        out_ref[:, :, 1:2] = fout.reshape(bb, t, 1)
        out_ref[:, :, 0:1] = jnp.zeros((bb, t, 1), jnp.float32)

    @pl.when(s > 0)
    def _backward():
        x = bwd_ref[...].reshape(m, -1)
        h = jnp.maximum(jnp.dot(x, w1, preferred_element_type=jnp.float32) + b1, 0.0)
        z = jnp.dot(h, w2, preferred_element_type=jnp.float32) + b2
        onehot = (jax.lax.broadcasted_iota(jnp.int32, (1, nact), 1) == s - 1)
        zsel = jnp.sum(jnp.where(onehot, z, 0.0), axis=1, keepdims=True)
        fin = _softplus(zsel).reshape(bb, t, 1)
        out_ref[:, :, 0:1] += fin


@functools.partial(jax.jit, static_argnames=("interpret",))
def _flow_pallas(forward_edges, backward_edges, W1, b1, W2, b2, interpret=False):
    b, t, a1, d = forward_edges.shape
    nact = a1 - 1
    h = W1.shape[1]
    bb = 4

    fwd = jnp.transpose(forward_edges, (0, 2, 1, 3))
    bwd = jnp.transpose(backward_edges, (0, 2, 1, 3))

    out = pl.pallas_call(
        functools.partial(_flow_body, nact=nact, bb=bb, t=t),
        grid=(b // bb, a1),
        in_specs=[
            pl.BlockSpec((bb, 1, t, d), lambda i, s: (i, 0, 0, 0)),
            pl.BlockSpec((bb, 1, t, d), lambda i, s: (i, jnp.maximum(s, 1), 0, 0)),
            pl.BlockSpec((d, h), lambda i, s: (0, 0)),
            pl.BlockSpec((1, h), lambda i, s: (0, 0)),
            pl.BlockSpec((h, nact), lambda i, s: (0, 0)),
            pl.BlockSpec((1, nact), lambda i, s: (0, 0)),
        ],
        out_specs=pl.BlockSpec((bb, t, 2), lambda i, s: (i, 0, 0)),
        out_shape=jax.ShapeDtypeStruct((b, t, 2), jnp.float32),
        compiler_params=pltpu.CompilerParams(
            dimension_semantics=("parallel", "arbitrary"),
        ),
        interpret=interpret,
    )(fwd, bwd, W1, b1.reshape(1, h), W2, b2.reshape(1, nact))
    return out


def kernel(forward_edges, backward_edges, paths_reward, W1, b1, W2, b2, initial_flow):
    b, t, a1, d = forward_edges.shape
    fin_fout = _flow_pallas(forward_edges, backward_edges, W1, b1, W2, b2)
    r = paths_reward.reshape(b, t, 1)
    finit = jnp.broadcast_to(initial_flow.reshape(1, 1, 1), (b, t, 1)).astype(jnp.float32)
    return jnp.concatenate([fin_fout, r, finit], axis=-1)


# speedup vs baseline: 3.3504x; 2.2500x over previous
import functools

import jax
import jax.numpy as jnp
from jax.experimental import pallas as pl
from jax.experimental.pallas import tpu as pltpu


def _softplus(x):
    return jnp.maximum(x, 0.0) + jnp.log1p(jnp.exp(-jnp.abs(x)))


def _flow_body(fwd_ref, bwd_ref, w1_ref, b1_ref, w2_ref, b2_ref, out_ref,
               *, nact, bb, t):
    m = bb * t
    w1 = w1_ref[...]
    b1 = b1_ref[...]
    w2 = w2_ref[...]
    b2 = b2_ref[...]

    x = fwd_ref[:, 0].reshape(m, -1)
    h = jnp.maximum(jnp.dot(x, w1, preferred_element_type=jnp.float32) + b1, 0.0)
    z = jnp.dot(h, w2, preferred_element_type=jnp.float32) + b2
    fout = jnp.sum(_softplus(z), axis=1, keepdims=True)

    fin = None
    for a in range(nact):
        x = bwd_ref[:, a + 1].reshape(m, -1)
        h = jnp.maximum(jnp.dot(x, w1, preferred_element_type=jnp.float32) + b1, 0.0)
        z = jnp.dot(h, w2, preferred_element_type=jnp.float32) + b2
        term = _softplus(z[:, a : a + 1])
        fin = term if fin is None else fin + term

    out_ref[:, :, 0:1] = fin.reshape(bb, t, 1)
    out_ref[:, :, 1:2] = fout.reshape(bb, t, 1)


@functools.partial(jax.jit, static_argnames=("interpret",))
def _flow_pallas(forward_edges, backward_edges, W1, b1, W2, b2, interpret=False):
    b, t, a1, d = forward_edges.shape
    nact = a1 - 1
    h = W1.shape[1]
    bb = 4

    fwd = jnp.transpose(forward_edges, (0, 2, 1, 3))
    bwd = jnp.transpose(backward_edges, (0, 2, 1, 3))

    out = pl.pallas_call(
        functools.partial(_flow_body, nact=nact, bb=bb, t=t),
        grid=(b // bb,),
        in_specs=[
            pl.BlockSpec((bb, 1, t, d), lambda i: (i, 0, 0, 0)),
            pl.BlockSpec((bb, a1, t, d), lambda i: (i, 0, 0, 0)),
            pl.BlockSpec((d, h), lambda i: (0, 0)),
            pl.BlockSpec((1, h), lambda i: (0, 0)),
            pl.BlockSpec((h, nact), lambda i: (0, 0)),
            pl.BlockSpec((1, nact), lambda i: (0, 0)),
        ],
        out_specs=pl.BlockSpec((bb, t, 2), lambda i: (i, 0, 0)),
        out_shape=jax.ShapeDtypeStruct((b, t, 2), jnp.float32),
        compiler_params=pltpu.CompilerParams(
            dimension_semantics=("parallel",),
        ),
        interpret=interpret,
    )(fwd, bwd, W1, b1.reshape(1, h), W2, b2.reshape(1, nact))
    return out


def kernel(forward_edges, backward_edges, paths_reward, W1, b1, W2, b2, initial_flow):
    b, t, a1, d = forward_edges.shape
    fin_fout = _flow_pallas(forward_edges, backward_edges, W1, b1, W2, b2)
    r = paths_reward.reshape(b, t, 1)
    finit = jnp.broadcast_to(initial_flow.reshape(1, 1, 1), (b, t, 1)).astype(jnp.float32)
    return jnp.concatenate([fin_fout, r, finit], axis=-1)
